# baseline (speedup 1.0000x reference)
import functools

import jax
import jax.numpy as jnp
from jax import lax
from jax.experimental import pallas as pl
from jax.experimental.pallas import tpu as pltpu
from jax.experimental.pallas import tpu_sc as plsc

N = 10000
E = 320000
IN = 128
C = 128
NEG_SLOPE = 0.2

NC = 2
NS = 16
L = 16

EDGES = E + N
EP = 331776
EW = EP // NS
EB = 1728
NBT = EW // EB
NB0 = 5
RB = 48
NGB = EB // RB
NPAD = 10112
SRO = 80
SCO = 128
ROWS_PER_TILE = NPAD // NS


def _mm_body(x_ref, w_ref, att_ref, h_ref, a_ref):
    h = jnp.dot(x_ref[...], w_ref[...],
                preferred_element_type=jnp.float32,
                precision=lax.Precision.HIGHEST)
    h_ref[...] = h
    a_ref[...] = jnp.dot(h, att_ref[...],
                         preferred_element_type=jnp.float32,
                         precision=lax.Precision.HIGHEST)


def _matmul(x, W, att2):
    blk = 1000
    grid = (N // blk,)
    return pl.pallas_call(
        _mm_body,
        grid=grid,
        in_specs=[
            pl.BlockSpec((blk, IN), lambda i: (i, 0)),
            pl.BlockSpec((IN, C), lambda i: (0, 0)),
            pl.BlockSpec((C, 8), lambda i: (0, 0)),
        ],
        out_specs=[
            pl.BlockSpec((blk, C), lambda i: (i, 0)),
            pl.BlockSpec((blk, 8), lambda i: (i, 0)),
        ],
        out_shape=[
            jax.ShapeDtypeStruct((N, C), jnp.float32),
            jax.ShapeDtypeStruct((N, 8), jnp.float32),
        ],
    )(x, W, att2)


def _ep_body(p_ref, s_ref, b_ref, o_ref):
    s = s_ref[:, 0] + s_ref[:, 1]
    v = (p_ref[0] + p_ref[1]) / (s[:, None] + 1e-16) + b_ref[...]
    o_ref[...] = jnp.where(v > 0, v, jnp.exp(jnp.minimum(v, 0.0)) - 1.0)


def _epilogue(parts, s2, bias):
    blk = 1000
    return pl.pallas_call(
        _ep_body,
        grid=(N // blk,),
        in_specs=[
            pl.BlockSpec((2, blk, C), lambda i: (0, i, 0)),
            pl.BlockSpec((blk, 2), lambda i: (i, 0)),
            pl.BlockSpec((1, C), lambda i: (0, 0)),
        ],
        out_specs=pl.BlockSpec((blk, C), lambda i: (i, 0)),
        out_shape=jax.ShapeDtypeStruct((N, C), jnp.float32),
    )(parts, s2, bias.reshape(1, C))


def _sc_body(h_hbm, asrc_hbm, adst_hbm, src_hbm, dst_hbm,
             out_hbm, s_hbm, ex_hbm,
             asrc_v, adst_v, s_v, src_v, dst_v, ex_v, rows_v, ridx_v,
             s_sh, out_sh, gsem0, gsem1, ssem0, ssem1):
    c = lax.axis_index("c")
    t = lax.axis_index("s")

    nb = jnp.where(c == 0, NB0, NBT - NB0)
    bstart = jnp.where(c == 0, 0, NB0)

    gsem = (gsem0, gsem1)
    ssem = (ssem0, ssem1)

    zero16 = jnp.zeros((L,), jnp.float32)
    iot = lax.iota(jnp.int32, L)

    def _zs(i, carry):
        for g in range(SCO // L):
            s_v[i, pl.ds(g * L, L)] = zero16
        return carry
    lax.fori_loop(0, SRO, _zs, 0)

    def _zb(i, carry):
        for g in range(C // L):
            rows_v[0, i, pl.ds(g * L, L)] = zero16
        return carry
    lax.fori_loop(0, RB, _zb, 0)

    @pl.when(t == 0)
    def _():
        pltpu.sync_copy(s_v, s_sh)

    zbase = t * ROWS_PER_TILE
    def _zo(k, carry):
        pltpu.sync_copy(rows_v.at[0],
                        out_sh.at[pl.ds(zbase + k * RB, RB), :])
        return carry
    lax.fori_loop(0, ROWS_PER_TILE // RB, _zo, 0)
    rem = ROWS_PER_TILE % RB
    pltpu.sync_copy(rows_v.at[0, pl.ds(0, rem), :],
                    out_sh.at[pl.ds(zbase + (ROWS_PER_TILE // RB) * RB, rem), :])

    pltpu.sync_copy(asrc_hbm, asrc_v.at[pl.ds(0, N)])
    pltpu.sync_copy(adst_hbm, adst_v.at[pl.ds(0, N)])
    asrc_v[pl.ds(N, L)] = zero16
    adst_v[pl.ds(N, L)] = zero16

    for g in range(SRO // L):
        ridx_v[pl.ds(g * L, L)] = iot + g * L

    plsc.subcore_barrier()

    def _pa_blk(bk, carry):
        off = t * EW + (bstart + bk) * EB
        pltpu.sync_copy(src_hbm.at[pl.ds(off, EB)], src_v)
        pltpu.sync_copy(dst_hbm.at[pl.ds(off, EB)], dst_v)

        def _pa(i, carry2):
            si = src_v[pl.ds(i * L, L)]
            dv = dst_v[pl.ds(i * L, L)]
            e = plsc.load_gather(asrc_v, [si]) + plsc.load_gather(adst_v, [dv])
            e = jnp.where(e >= 0, e, e * NEG_SLOPE)
            ex = jnp.exp(e)
            ex_v[pl.ds(i * L, L)] = ex
            plsc.addupdate_scatter(
                s_v, [jnp.right_shift(dv, 7), jnp.bitwise_and(dv, SCO - 1)], ex)
            return carry2
        lax.fori_loop(0, EB // L, _pa, 0)
        pltpu.sync_copy(ex_v, ex_hbm.at[pl.ds(off, EB)])
        return carry
    lax.fori_loop(0, nb, _pa_blk, 0)

    pltpu.sync_copy(s_v, s_sh.at[ridx_v], add=True)

    plsc.subcore_barrier()

    @pl.when(t == 0)
    def _():
        pltpu.sync_copy(s_sh, s_hbm.at[c])

    def _fire_gather(g, b):
        pltpu.async_copy(
            h_hbm.at[src_v.at[pl.ds(g * RB, RB)]], rows_v.at[b], gsem[b])

    def _wait_gather(b):
        pltpu.make_async_copy(
            h_hbm.at[src_v.at[pl.ds(0, RB)]], rows_v.at[b], gsem[b]).wait()

    def _fire_scatter(g, b):
        pltpu.async_copy(
            rows_v.at[b], out_sh.at[dst_v.at[pl.ds(g * RB, RB)]],
            ssem[b], add=True)

    def _wait_scatter(b):
        pltpu.make_async_copy(
            rows_v.at[b], out_sh.at[dst_v.at[pl.ds(0, RB)]], ssem[b]).wait()

    def _scale(g, b):
        def _sub(j, carry):
            i16 = g * (RB // L) + j
            exv = ex_v[pl.ds(i16 * L, L)]
            for rr in range(L):
                av = exv.at[jnp.full((L,), rr, jnp.int32)].get(
                    mode="promise_in_bounds")
                row = j * L + rr
                for gg in range(C // L):
                    rows_v[b, row, pl.ds(gg * L, L)] = (
                        rows_v[b, row, pl.ds(gg * L, L)] * av)
            return carry
        lax.fori_loop(0, RB // L, _sub, 0)

    def _pb_blk(bk, carry):
        off = t * EW + (bstart + bk) * EB
        pltpu.sync_copy(src_hbm.at[pl.ds(off, EB)], src_v)
        pltpu.sync_copy(dst_hbm.at[pl.ds(off, EB)], dst_v)
        pltpu.sync_copy(ex_hbm.at[pl.ds(off, EB)], ex_v)

        _fire_gather(0, 0)

        def _pb_outer(o, carry2):
            for b in range(2):
                g = o * 2 + b
                @pl.when(g >= 1)
                def _():
                    _wait_scatter(1 - b)
                @pl.when(g + 1 < NGB)
                def _():
                    _fire_gather(g + 1, 1 - b)
                _wait_gather(b)
                _scale(g, b)
                _fire_scatter(g, b)
            return carry2
        lax.fori_loop(0, NGB // 2, _pb_outer, 0)
        _wait_scatter((NGB - 1) % 2)
        return carry
    lax.fori_loop(0, nb, _pb_blk, 0)

    plsc.subcore_barrier()

    pltpu.sync_copy(out_sh.at[pl.ds(t * ROWS_PER_TILE, ROWS_PER_TILE), :],
                    out_hbm.at[c, pl.ds(t * ROWS_PER_TILE, ROWS_PER_TILE), :])


def _sc_edges(h, a_src, a_dst, src2, dst2):
    mesh = plsc.VectorSubcoreMesh(core_axis_name="c", subcore_axis_name="s")
    fn = pl.kernel(
        _sc_body,
        (jax.ShapeDtypeStruct((NC, NPAD, C), jnp.float32),
         jax.ShapeDtypeStruct((NC, SRO, SCO), jnp.float32),
         jax.ShapeDtypeStruct((EP,), jnp.float32)),
        mesh=mesh,
        compiler_params=pltpu.CompilerParams(needs_layout_passes=False),
        scratch_types=[
            pltpu.VMEM((NPAD,), jnp.float32),
            pltpu.VMEM((NPAD,), jnp.float32),
            pltpu.VMEM((SRO, SCO), jnp.float32),
            pltpu.VMEM((EB,), jnp.int32),
            pltpu.VMEM((EB,), jnp.int32),
            pltpu.VMEM((EB,), jnp.float32),
            pltpu.VMEM((2, RB, C), jnp.float32),
            pltpu.VMEM((SRO,), jnp.int32),
            pltpu.VMEM_SHARED((SRO, SCO), jnp.float32),
            pltpu.VMEM_SHARED((NPAD, C), jnp.float32),
            pltpu.SemaphoreType.DMA,
            pltpu.SemaphoreType.DMA,
            pltpu.SemaphoreType.DMA,
            pltpu.SemaphoreType.DMA,
        ],
    )
    return fn(h, a_src, a_dst, src2, dst2)


def kernel(x, edge_index, W, att_src, att_dst, bias):
    loop = jnp.arange(N, dtype=jnp.int32)
    pad = EP - EDGES
    src2 = jnp.concatenate(
        [edge_index[0], loop, jnp.zeros((pad,), jnp.int32)])
    dst2 = jnp.concatenate(
        [edge_index[1], loop, jnp.full((pad,), N, jnp.int32)])

    att2 = jnp.concatenate(
        [att_src.reshape(C, 1), att_dst.reshape(C, 1),
         jnp.zeros((C, 6), jnp.float32)], axis=1)

    h, a = _matmul(x, W, att2)
    a_src = a[:, 0]
    a_dst = a[:, 1]

    parts, s2, _ = _sc_edges(h, a_src, a_dst, src2, dst2)
    s2 = s2.reshape(NC, SRO * SCO)[:, :N].T
    return _epilogue(parts, s2, bias)


# speedup vs baseline: 1.1324x; 1.1324x over previous
import functools

import jax
import jax.numpy as jnp
from jax import lax
from jax.experimental import pallas as pl
from jax.experimental.pallas import tpu as pltpu
from jax.experimental.pallas import tpu_sc as plsc

N = 10000
E = 320000
IN = 128
C = 128
NEG_SLOPE = 0.2

NC = 2
NS = 16
L = 16

EDGES = E + N
EP = 331776
EW = EP // NS
EB = 1728
NBT = EW // EB
NB0 = 7
RB = 48
NGB = EB // RB
NPAD = 10112
SRO = 80
SCO = 128
ROWS_PER_TILE = NPAD // NS


def _mm_body(x_ref, w_ref, att_ref, h_ref, a_ref):
    h = jnp.dot(x_ref[...], w_ref[...],
                preferred_element_type=jnp.float32,
                precision=lax.Precision.HIGHEST)
    h_ref[...] = h
    a_ref[...] = jnp.dot(h, att_ref[...],
                         preferred_element_type=jnp.float32,
                         precision=lax.Precision.HIGHEST)


def _matmul(x, W, att2):
    blk = 1000
    grid = (N // blk,)
    return pl.pallas_call(
        _mm_body,
        grid=grid,
        in_specs=[
            pl.BlockSpec((blk, IN), lambda i: (i, 0)),
            pl.BlockSpec((IN, C), lambda i: (0, 0)),
            pl.BlockSpec((C, 8), lambda i: (0, 0)),
        ],
        out_specs=[
            pl.BlockSpec((blk, C), lambda i: (i, 0)),
            pl.BlockSpec((blk, 8), lambda i: (i, 0)),
        ],
        out_shape=[
            jax.ShapeDtypeStruct((N, C), jnp.float32),
            jax.ShapeDtypeStruct((N, 8), jnp.float32),
        ],
    )(x, W, att2)


def _ep_body(p_ref, s_ref, b_ref, o_ref):
    s = s_ref[:, 0] + s_ref[:, 1]
    v = (p_ref[0] + p_ref[1]) / (s[:, None] + 1e-16) + b_ref[...]
    o_ref[...] = jnp.where(v > 0, v, jnp.exp(jnp.minimum(v, 0.0)) - 1.0)


def _epilogue(parts, s2, bias):
    blk = 1000
    return pl.pallas_call(
        _ep_body,
        grid=(N // blk,),
        in_specs=[
            pl.BlockSpec((2, blk, C), lambda i: (0, i, 0)),
            pl.BlockSpec((blk, 2), lambda i: (i, 0)),
            pl.BlockSpec((1, C), lambda i: (0, 0)),
        ],
        out_specs=pl.BlockSpec((blk, C), lambda i: (i, 0)),
        out_shape=jax.ShapeDtypeStruct((N, C), jnp.float32),
    )(parts, s2, bias.reshape(1, C))


def _sc_body(h_hbm, asrc_hbm, adst_hbm, src_hbm, dst_hbm,
             out_hbm, s_hbm, ex_hbm,
             asrc_v, adst_v, s_v, src_v, dst_v, ex_v, rows_v, ridx_v,
             s_sh, out_sh, gsem0, gsem1, ssem0, ssem1):
    c = lax.axis_index("c")
    t = lax.axis_index("s")

    nb = jnp.where(c == 0, NB0, NBT - NB0)
    bstart = jnp.where(c == 0, 0, NB0)

    gsem = (gsem0, gsem1)
    ssem = (ssem0, ssem1)

    zero16 = jnp.zeros((L,), jnp.float32)
    iot = lax.iota(jnp.int32, L)

    def _zs(i, carry):
        for g in range(SCO // L):
            s_v[i, pl.ds(g * L, L)] = zero16
        return carry
    lax.fori_loop(0, SRO, _zs, 0)

    def _zb(i, carry):
        for g in range(C // L):
            rows_v[0, i, pl.ds(g * L, L)] = zero16
        return carry
    lax.fori_loop(0, RB, _zb, 0)

    @pl.when(t == 0)
    def _():
        pltpu.sync_copy(s_v, s_sh)

    zbase = t * ROWS_PER_TILE
    def _zo(k, carry):
        pltpu.sync_copy(rows_v.at[0],
                        out_sh.at[pl.ds(zbase + k * RB, RB), :])
        return carry
    lax.fori_loop(0, ROWS_PER_TILE // RB, _zo, 0)
    rem = ROWS_PER_TILE % RB
    pltpu.sync_copy(rows_v.at[0, pl.ds(0, rem), :],
                    out_sh.at[pl.ds(zbase + (ROWS_PER_TILE // RB) * RB, rem), :])

    pltpu.sync_copy(asrc_hbm, asrc_v.at[pl.ds(0, N)])
    pltpu.sync_copy(adst_hbm, adst_v.at[pl.ds(0, N)])
    asrc_v[pl.ds(N, L)] = zero16
    adst_v[pl.ds(N, L)] = zero16

    for g in range(SRO // L):
        ridx_v[pl.ds(g * L, L)] = iot + g * L

    plsc.subcore_barrier()

    def _pa_blk(bk, carry):
        off = t * EW + (bstart + bk) * EB
        pltpu.sync_copy(src_hbm.at[pl.ds(off, EB)], src_v)
        pltpu.sync_copy(dst_hbm.at[pl.ds(off, EB)], dst_v)

        def _pa(i, carry2):
            si = src_v[pl.ds(i * L, L)]
            dv = dst_v[pl.ds(i * L, L)]
            e = plsc.load_gather(asrc_v, [si]) + plsc.load_gather(adst_v, [dv])
            e = jnp.where(e >= 0, e, e * NEG_SLOPE)
            ex = jnp.exp(e)
            ex_v[pl.ds(i * L, L)] = ex
            plsc.addupdate_scatter(
                s_v, [jnp.right_shift(dv, 7), jnp.bitwise_and(dv, SCO - 1)], ex)
            return carry2
        lax.fori_loop(0, EB // L, _pa, 0)
        pltpu.sync_copy(ex_v, ex_hbm.at[pl.ds(off, EB)])
        return carry
    lax.fori_loop(0, nb, _pa_blk, 0)

    pltpu.sync_copy(s_v, s_sh.at[ridx_v], add=True)

    plsc.subcore_barrier()

    @pl.when(t == 0)
    def _():
        pltpu.sync_copy(s_sh, s_hbm.at[c])

    def _fire_gather(g, b):
        pltpu.async_copy(
            h_hbm.at[src_v.at[pl.ds(g * RB, RB)]], rows_v.at[b], gsem[b])

    def _wait_gather(b):
        pltpu.make_async_copy(
            h_hbm.at[src_v.at[pl.ds(0, RB)]], rows_v.at[b], gsem[b]).wait()

    def _fire_scatter(g, b):
        pltpu.async_copy(
            rows_v.at[b], out_sh.at[dst_v.at[pl.ds(g * RB, RB)]],
            ssem[b], add=True)

    def _wait_scatter(b):
        pltpu.make_async_copy(
            rows_v.at[b], out_sh.at[dst_v.at[pl.ds(0, RB)]], ssem[b]).wait()

    def _scale(g, b):
        def _sub(j, carry):
            i16 = g * (RB // L) + j
            exv = ex_v[pl.ds(i16 * L, L)]
            for rr in range(L):
                av = exv.at[jnp.full((L,), rr, jnp.int32)].get(
                    mode="promise_in_bounds")
                row = j * L + rr
                for gg in range(C // L):
                    rows_v[b, row, pl.ds(gg * L, L)] = (
                        rows_v[b, row, pl.ds(gg * L, L)] * av)
            return carry
        lax.fori_loop(0, RB // L, _sub, 0)

    def _pb_blk(bk, carry):
        off = t * EW + (bstart + bk) * EB
        pltpu.sync_copy(src_hbm.at[pl.ds(off, EB)], src_v)
        pltpu.sync_copy(dst_hbm.at[pl.ds(off, EB)], dst_v)
        pltpu.sync_copy(ex_hbm.at[pl.ds(off, EB)], ex_v)

        _fire_gather(0, 0)

        def _pb_outer(o, carry2):
            for b in range(2):
                g = o * 2 + b
                @pl.when(g >= 1)
                def _():
                    _wait_scatter(1 - b)
                @pl.when(g + 1 < NGB)
                def _():
                    _fire_gather(g + 1, 1 - b)
                _wait_gather(b)
                _scale(g, b)
                _fire_scatter(g, b)
            return carry2
        lax.fori_loop(0, NGB // 2, _pb_outer, 0)
        _wait_scatter((NGB - 1) % 2)
        return carry
    lax.fori_loop(0, nb, _pb_blk, 0)

    plsc.subcore_barrier()

    pltpu.sync_copy(out_sh.at[pl.ds(t * ROWS_PER_TILE, ROWS_PER_TILE), :],
                    out_hbm.at[c, pl.ds(t * ROWS_PER_TILE, ROWS_PER_TILE), :])


def _sc_edges(h, a_src, a_dst, src2, dst2):
    mesh = plsc.VectorSubcoreMesh(core_axis_name="c", subcore_axis_name="s")
    fn = pl.kernel(
        _sc_body,
        (jax.ShapeDtypeStruct((NC, NPAD, C), jnp.float32),
         jax.ShapeDtypeStruct((NC, SRO, SCO), jnp.float32),
         jax.ShapeDtypeStruct((EP,), jnp.float32)),
        mesh=mesh,
        compiler_params=pltpu.CompilerParams(needs_layout_passes=False),
        scratch_types=[
            pltpu.VMEM((NPAD,), jnp.float32),
            pltpu.VMEM((NPAD,), jnp.float32),
            pltpu.VMEM((SRO, SCO), jnp.float32),
            pltpu.VMEM((EB,), jnp.int32),
            pltpu.VMEM((EB,), jnp.int32),
            pltpu.VMEM((EB,), jnp.float32),
            pltpu.VMEM((2, RB, C), jnp.float32),
            pltpu.VMEM((SRO,), jnp.int32),
            pltpu.VMEM_SHARED((SRO, SCO), jnp.float32),
            pltpu.VMEM_SHARED((NPAD, C), jnp.float32),
            pltpu.SemaphoreType.DMA,
            pltpu.SemaphoreType.DMA,
            pltpu.SemaphoreType.DMA,
            pltpu.SemaphoreType.DMA,
        ],
    )
    return fn(h, a_src, a_dst, src2, dst2)


def kernel(x, edge_index, W, att_src, att_dst, bias):
    loop = jnp.arange(N, dtype=jnp.int32)
    pad = EP - EDGES
    src2 = jnp.concatenate(
        [edge_index[0], loop, jnp.zeros((pad,), jnp.int32)])
    dst2 = jnp.concatenate(
        [edge_index[1], loop, jnp.full((pad,), N, jnp.int32)])

    att2 = jnp.concatenate(
        [att_src.reshape(C, 1), att_dst.reshape(C, 1),
         jnp.zeros((C, 6), jnp.float32)], axis=1)

    h, a = _matmul(x, W, att2)
    a_src = a[:, 0]
    a_dst = a[:, 1]

    parts, s2, _ = _sc_edges(h, a_src, a_dst, src2, dst2)
    s2 = s2.reshape(NC, SRO * SCO)[:, :N].T
    return _epilogue(parts, s2, bias)
